# baseline (speedup 1.0000x reference)
import functools

import jax
import jax.numpy as jnp
from jax import lax
from jax.experimental import pallas as pl
from jax.experimental.pallas import tpu as pltpu
from jax.experimental.pallas import tpu_sc as plsc

NC = 2
NS = 16
NW = NC * NS
L = 16

CH = 16384
NB1 = 1 << 16
NB2 = 1 << 15

SIG_P = 0.9933071490757153
SIG_N = 0.006692850924284856

_MASK31 = 0x7FFFFFFF
_MASK15 = 0x7FFF


def _sc_hist_body(tv_hbm, b1_hbm, out_hbm, d0, d1, hist, b1buf, s0, s1,
                  *, nbuckets, per_w, nch, pass2, CH):
    cid = lax.axis_index("c")
    sid = lax.axis_index("s")
    wid = cid * NS + sid
    base = wid * per_w

    zero16 = jnp.zeros((L,), jnp.int32)
    ones16 = jnp.ones((L,), jnp.int32)

    @plsc.parallel_loop(0, nbuckets, step=L, unroll=8)
    def _(i):
        hist[pl.ds(i, L)] = zero16

    if pass2:
        pltpu.sync_copy(b1_hbm.at[pl.ds(0, L)], b1buf)
        b1v = b1buf[...]

    def start(c, buf, sem):
        pltpu.make_async_copy(
            tv_hbm.at[pl.ds(base + c * CH, CH)], buf, sem).start()

    def wait(c, buf, sem):
        pltpu.make_async_copy(
            tv_hbm.at[pl.ds(base + c * CH, CH)], buf, sem).wait()

    def process(buf):
        @plsc.parallel_loop(0, CH, step=L, unroll=8)
        def _(j):
            v = buf[pl.ds(j, L)]
            bits = lax.bitcast_convert_type(v, jnp.int32)
            m = jnp.bitwise_and(bits, _MASK31)
            if pass2:
                lo = jnp.bitwise_and(m, _MASK15)
                hi = jnp.right_shift(m, 15)
                plsc.addupdate_scatter(hist, [lo], ones16,
                                       mask=hi == b1v)
            else:
                hi = jnp.right_shift(m, 15)
                plsc.addupdate_scatter(hist, [hi], ones16)

    start(0, d0, s0)
    start(1, d1, s1)
    bufs = ((d0, s0), (d1, s1))

    def outer(g, _):
        for b in range(2):
            c = g * 2 + b
            buf, sem = bufs[b]
            wait(c, buf, sem)
            process(buf)

            @pl.when(c + 2 < nch)
            def _():
                start(c + 2, buf, sem)
        return 0
    lax.fori_loop(0, nch // 2, outer, 0)

    pltpu.sync_copy(hist, out_hbm.at[wid])


def _make_sc_hist(n, nbuckets, pass2, ch):
    per_w = n // NW
    nch = per_w // ch
    mesh = plsc.VectorSubcoreMesh(core_axis_name="c", subcore_axis_name="s",
                                  num_cores=NC, num_subcores=NS)
    body = functools.partial(_sc_hist_body, nbuckets=nbuckets, per_w=per_w,
                             nch=nch, pass2=pass2, CH=ch)

    def wrapped(tv, b1_flat):
        return pl.kernel(
            body,
            out_type=jax.ShapeDtypeStruct((NW, nbuckets), jnp.int32),
            mesh=mesh,
            compiler_params=pltpu.CompilerParams(needs_layout_passes=False),
            scratch_types=[
                pltpu.VMEM((ch,), jnp.float32),
                pltpu.VMEM((ch,), jnp.float32),
                pltpu.VMEM((nbuckets,), jnp.int32),
                pltpu.VMEM((L,), jnp.int32),
                pltpu.SemaphoreType.DMA,
                pltpu.SemaphoreType.DMA,
            ],
        )(tv, b1_flat)
    return wrapped


def _suffix_counts(h2d, rows, cols):
    rowtot = jnp.sum(h2d, axis=1).reshape(1, rows)
    ri = lax.broadcasted_iota(jnp.int32, (rows, rows), 0)
    rj = lax.broadcasted_iota(jnp.int32, (rows, rows), 1)
    ur = (ri > rj).astype(jnp.float32)
    rsuf = lax.dot_general(rowtot, ur, (((1,), (0,)), ((), ())),
                           precision=lax.Precision.HIGHEST,
                           preferred_element_type=jnp.float32)
    ci = lax.broadcasted_iota(jnp.int32, (cols, cols), 0)
    cj = lax.broadcasted_iota(jnp.int32, (cols, cols), 1)
    uc = (ci > cj).astype(jnp.float32)
    csuf = lax.dot_general(h2d, uc, (((1,), (0,)), ((), ())),
                           precision=lax.Precision.HIGHEST,
                           preferred_element_type=jnp.float32)
    return rsuf.reshape(rows, 1) + csuf


def _select1_body(hist_ref, k_ref, outb_ref, outk_ref):
    rows, cols = 512, 128
    h = jnp.sum(hist_ref[...].astype(jnp.float32), axis=0).reshape(rows, cols)
    s = _suffix_counts(h, rows, cols)
    kf = k_ref[0, 0].astype(jnp.float32)
    found = jnp.logical_and(s < kf, kf <= s + h)
    bidx = (lax.broadcasted_iota(jnp.int32, (rows, cols), 0) * cols
            + lax.broadcasted_iota(jnp.int32, (rows, cols), 1))
    b1 = jnp.max(jnp.where(found, bidx, -1))
    sb = jnp.max(jnp.where(found, s, 0.0))
    kp = k_ref[0, 0] - sb.astype(jnp.int32)
    outb_ref[...] = jnp.full((1, 128), b1, jnp.int32)
    outk_ref[...] = jnp.full((1, 128), kp, jnp.int32)


def _select2_mask_body(hist_ref, kp_ref, b1_ref, tv_ref, out_ref, thr_ref):
    @pl.when(pl.program_id(0) == 0)
    def _():
        rows, cols = 256, 128
        h = jnp.sum(hist_ref[...].astype(jnp.float32), axis=0).reshape(rows,
                                                                       cols)
        s = _suffix_counts(h, rows, cols)
        kf = kp_ref[0, 0].astype(jnp.float32)
        found = jnp.logical_and(s < kf, kf <= s + h)
        bidx = (lax.broadcasted_iota(jnp.int32, (rows, cols), 0) * cols
                + lax.broadcasted_iota(jnp.int32, (rows, cols), 1))
        b2 = jnp.max(jnp.where(found, bidx, -1))
        bits = jnp.bitwise_or(jnp.left_shift(b1_ref[0, 0], 15), b2)
        thr_ref[0, 0] = lax.bitcast_convert_type(bits, jnp.float32)

    t = thr_ref[0, 0]
    x = tv_ref[...]
    out_ref[...] = x * jnp.where(jnp.abs(x) > t,
                                 jnp.float32(SIG_P), jnp.float32(SIG_N))


def _tc_select1(hist1, k2d):
    return pl.pallas_call(
        _select1_body,
        in_specs=[pl.BlockSpec(memory_space=pltpu.VMEM),
                  pl.BlockSpec(memory_space=pltpu.SMEM)],
        out_specs=[pl.BlockSpec(memory_space=pltpu.VMEM),
                   pl.BlockSpec(memory_space=pltpu.VMEM)],
        out_shape=[jax.ShapeDtypeStruct((1, 128), jnp.int32),
                   jax.ShapeDtypeStruct((1, 128), jnp.int32)],
    )(hist1, k2d)


def _tc_select2_mask(hist2, outk, outb, tv, blk):
    n = tv.shape[0]
    return pl.pallas_call(
        _select2_mask_body,
        grid=(n // blk,),
        in_specs=[pl.BlockSpec((NW, NB2), lambda i: (0, 0)),
                  pl.BlockSpec(memory_space=pltpu.SMEM),
                  pl.BlockSpec(memory_space=pltpu.SMEM),
                  pl.BlockSpec((blk,), lambda i: (i,))],
        out_specs=pl.BlockSpec((blk,), lambda i: (i,)),
        out_shape=jax.ShapeDtypeStruct((n,), jnp.float32),
        scratch_shapes=[pltpu.SMEM((1, 1), jnp.float32)],
    )(hist2, outk, outb, tv)


def kernel(tv, k):
    n = tv.shape[0]
    assert n % (NW * 2 * CH) == 0

    sc_hist1 = _make_sc_hist(n, NB1, pass2=False, ch=CH)
    sc_hist2 = _make_sc_hist(n, NB2, pass2=True, ch=2 * CH)

    dummy_b1 = jnp.zeros((128,), jnp.int32)
    hist1 = sc_hist1(tv, dummy_b1)

    k2d = jnp.asarray(k, jnp.int32).reshape(1, 1)
    outb, outk = _tc_select1(hist1, k2d)

    hist2 = sc_hist2(tv, outb.reshape(128))
    return _tc_select2_mask(hist2, outk, outb, tv, n // 32)


# speedup vs baseline: 1.0358x; 1.0358x over previous
import functools

import jax
import jax.numpy as jnp
from jax import lax
from jax.experimental import pallas as pl
from jax.experimental.pallas import tpu as pltpu
from jax.experimental.pallas import tpu_sc as plsc

NC = 2
NS = 16
NW = NC * NS
L = 16

CH = 16384
NB1 = 1 << 16
NB2 = 1 << 15

SIG_P = 0.9933071490757153
SIG_N = 0.006692850924284856

_MASK31 = 0x7FFFFFFF
_MASK15 = 0x7FFF


def _sc_hist_body(tv_hbm, b1_hbm, out_hbm, d0, d1, hist, b1buf, s0, s1,
                  *, nbuckets, per_w, nch, pass2, CH):
    cid = lax.axis_index("c")
    sid = lax.axis_index("s")
    wid = cid * NS + sid
    base = wid * per_w

    zero16 = jnp.zeros((L,), jnp.int32)
    ones16 = jnp.ones((L,), jnp.int32)

    @plsc.parallel_loop(0, nbuckets, step=L, unroll=8)
    def _(i):
        hist[pl.ds(i, L)] = zero16

    if pass2:
        pltpu.sync_copy(b1_hbm.at[pl.ds(0, L)], b1buf)
        b1v = b1buf[...]

    def start(c, buf, sem):
        pltpu.make_async_copy(
            tv_hbm.at[pl.ds(base + c * CH, CH)], buf, sem).start()

    def wait(c, buf, sem):
        pltpu.make_async_copy(
            tv_hbm.at[pl.ds(base + c * CH, CH)], buf, sem).wait()

    def process(buf):
        @plsc.parallel_loop(0, CH, step=L, unroll=8)
        def _(j):
            v = buf[pl.ds(j, L)]
            bits = lax.bitcast_convert_type(v, jnp.int32)
            m = jnp.bitwise_and(bits, _MASK31)
            if pass2:
                lo = jnp.bitwise_and(m, _MASK15)
                hi = jnp.right_shift(m, 15)
                plsc.addupdate_scatter(hist, [lo], ones16,
                                       mask=hi == b1v)
            else:
                hi = jnp.right_shift(m, 15)
                plsc.addupdate_scatter(hist, [hi], ones16)

    start(0, d0, s0)
    start(1, d1, s1)
    bufs = ((d0, s0), (d1, s1))

    def outer(g, _):
        for b in range(2):
            c = g * 2 + b
            buf, sem = bufs[b]
            wait(c, buf, sem)
            process(buf)

            @pl.when(c + 2 < nch)
            def _():
                start(c + 2, buf, sem)
        return 0
    lax.fori_loop(0, nch // 2, outer, 0)

    pltpu.sync_copy(hist, out_hbm.at[wid])


def _make_sc_hist(n, nbuckets, pass2, ch):
    per_w = n // NW
    nch = per_w // ch
    mesh = plsc.VectorSubcoreMesh(core_axis_name="c", subcore_axis_name="s",
                                  num_cores=NC, num_subcores=NS)
    body = functools.partial(_sc_hist_body, nbuckets=nbuckets, per_w=per_w,
                             nch=nch, pass2=pass2, CH=ch)

    def wrapped(tv, b1_flat):
        return pl.kernel(
            body,
            out_type=jax.ShapeDtypeStruct((NW, nbuckets), jnp.int32),
            mesh=mesh,
            compiler_params=pltpu.CompilerParams(needs_layout_passes=False),
            scratch_types=[
                pltpu.VMEM((ch,), jnp.float32),
                pltpu.VMEM((ch,), jnp.float32),
                pltpu.VMEM((nbuckets,), jnp.int32),
                pltpu.VMEM((L,), jnp.int32),
                pltpu.SemaphoreType.DMA,
                pltpu.SemaphoreType.DMA,
            ],
        )(tv, b1_flat)
    return wrapped


def _suffix_counts(h2d, rows, cols):
    rowtot = jnp.sum(h2d, axis=1).reshape(1, rows)
    ri = lax.broadcasted_iota(jnp.int32, (rows, rows), 0)
    rj = lax.broadcasted_iota(jnp.int32, (rows, rows), 1)
    ur = (ri > rj).astype(jnp.float32)
    rsuf = lax.dot_general(rowtot, ur, (((1,), (0,)), ((), ())),
                           precision=lax.Precision.HIGHEST,
                           preferred_element_type=jnp.float32)
    ci = lax.broadcasted_iota(jnp.int32, (cols, cols), 0)
    cj = lax.broadcasted_iota(jnp.int32, (cols, cols), 1)
    uc = (ci > cj).astype(jnp.float32)
    csuf = lax.dot_general(h2d, uc, (((1,), (0,)), ((), ())),
                           precision=lax.Precision.HIGHEST,
                           preferred_element_type=jnp.float32)
    return rsuf.reshape(rows, 1) + csuf


def _select1_body(hist_ref, k_ref, outb_ref, outk_ref):
    rows, cols = 512, 128
    h = jnp.sum(hist_ref[...].astype(jnp.float32), axis=0).reshape(rows, cols)
    s = _suffix_counts(h, rows, cols)
    kf = k_ref[0, 0].astype(jnp.float32)
    found = jnp.logical_and(s < kf, kf <= s + h)
    bidx = (lax.broadcasted_iota(jnp.int32, (rows, cols), 0) * cols
            + lax.broadcasted_iota(jnp.int32, (rows, cols), 1))
    b1 = jnp.max(jnp.where(found, bidx, -1))
    sb = jnp.max(jnp.where(found, s, 0.0))
    kp = k_ref[0, 0] - sb.astype(jnp.int32)
    outb_ref[...] = jnp.full((1, 128), b1, jnp.int32)
    outk_ref[...] = jnp.full((1, 128), kp, jnp.int32)


def _select2_mask_body(hist_ref, kp_ref, b1_ref, tv_ref, out_ref, thr_ref):
    @pl.when(pl.program_id(0) == 0)
    def _():
        rows, cols = 256, 128
        h = jnp.sum(hist_ref[...].astype(jnp.float32), axis=0).reshape(rows,
                                                                       cols)
        s = _suffix_counts(h, rows, cols)
        kf = kp_ref[0, 0].astype(jnp.float32)
        found = jnp.logical_and(s < kf, kf <= s + h)
        bidx = (lax.broadcasted_iota(jnp.int32, (rows, cols), 0) * cols
                + lax.broadcasted_iota(jnp.int32, (rows, cols), 1))
        b2 = jnp.max(jnp.where(found, bidx, -1))
        bits = jnp.bitwise_or(jnp.left_shift(b1_ref[0, 0], 15), b2)
        thr_ref[0, 0] = lax.bitcast_convert_type(bits, jnp.float32)

    t = thr_ref[0, 0]
    x = tv_ref[...]
    out_ref[...] = x * jnp.where(jnp.abs(x) > t,
                                 jnp.float32(SIG_P), jnp.float32(SIG_N))


def _tc_select1(hist1, k2d):
    return pl.pallas_call(
        _select1_body,
        in_specs=[pl.BlockSpec(memory_space=pltpu.VMEM),
                  pl.BlockSpec(memory_space=pltpu.SMEM)],
        out_specs=[pl.BlockSpec(memory_space=pltpu.VMEM),
                   pl.BlockSpec(memory_space=pltpu.VMEM)],
        out_shape=[jax.ShapeDtypeStruct((1, 128), jnp.int32),
                   jax.ShapeDtypeStruct((1, 128), jnp.int32)],
    )(hist1, k2d)


def _tc_select2_mask(hist2, outk, outb, tv, blk):
    n = tv.shape[0]
    return pl.pallas_call(
        _select2_mask_body,
        grid=(n // blk,),
        in_specs=[pl.BlockSpec((NW, NB2), lambda i: (0, 0)),
                  pl.BlockSpec(memory_space=pltpu.SMEM),
                  pl.BlockSpec(memory_space=pltpu.SMEM),
                  pl.BlockSpec((blk,), lambda i: (i,))],
        out_specs=pl.BlockSpec((blk,), lambda i: (i,)),
        out_shape=jax.ShapeDtypeStruct((n,), jnp.float32),
        scratch_shapes=[pltpu.SMEM((1, 1), jnp.float32)],
    )(hist2, outk, outb, tv)


def kernel(tv, k):
    n = tv.shape[0]
    assert n % (NW * 2 * CH) == 0

    sc_hist1 = _make_sc_hist(n, NB1, pass2=False, ch=CH)
    sc_hist2 = _make_sc_hist(n, NB2, pass2=True, ch=2 * CH)

    dummy_b1 = jnp.zeros((128,), jnp.int32)
    hist1 = sc_hist1(tv, dummy_b1)

    k2d = jnp.asarray(k, jnp.int32).reshape(1, 1)
    outb, outk = _tc_select1(hist1, k2d)

    hist2 = sc_hist2(tv, outb.reshape(128))
    return _tc_select2_mask(hist2, outk, outb, tv, n // 8)
